# Initial kernel scaffold; baseline (speedup 1.0000x reference)
import jax
import jax.numpy as jnp
from jax.experimental import pallas as pl


def kernel(x, weights, biases, out_idx, out_sign, max_iters):
    raise NotImplementedError("write your pallas kernel here")


# speedup vs baseline: 3.7264x; 3.7264x over previous
import functools

import jax
import jax.numpy as jnp
from jax import lax
from jax.experimental import pallas as pl
from jax.experimental.pallas import tpu as pltpu
from jax.experimental.pallas import tpu_sc as plsc

_SC_PARAMS = pltpu.CompilerParams(
    needs_layout_passes=False, use_tc_tiling_on_sc=False
)

NC = 2
NS = 16
NW = NC * NS
L = 16
CH = 128


def _sc_gather(w16, out_idx, x16, R, S):
    rows_per = R // NW
    nch = R // CH
    SH = S // 16

    mesh = plsc.VectorSubcoreMesh(core_axis_name="c", subcore_axis_name="s")

    @functools.partial(
        pl.kernel,
        out_type=(
            jax.ShapeDtypeStruct((R, R), jnp.float32),
            jax.ShapeDtypeStruct((R,), jnp.float32),
        ),
        mesh=mesh,
        compiler_params=_SC_PARAMS,
        scratch_types=[
            pltpu.VMEM((nch, CH), jnp.int32),
            pltpu.VMEM((R,), jnp.int32),
            pltpu.VMEM((nch, CH), jnp.int32),
            pltpu.VMEM((R, 16), jnp.float32),
            pltpu.VMEM((rows_per, R), jnp.float32),
            pltpu.VMEM((R,), jnp.float32),
            pltpu.SemaphoreType.DMA,
        ],
    )
    def k(w_hbm, oi_hbm, x_hbm, wsub_hbm, xsub_hbm,
          oih_v, oil_v, gi_v, grow_v, rows_v, xs_v, sem):
        wid = lax.axis_index("s") * NC + lax.axis_index("c")
        row0 = wid * rows_per
        for c in range(nch):
            pltpu.sync_copy(oi_hbm.at[pl.ds(c * CH, CH)], gi_v.at[c])
        for c in range(nch):
            for t in range(CH // L):
                raw = gi_v[c, pl.ds(t * L, L)]
                oih_v[c, pl.ds(t * L, L)] = raw >> 4
                oil_v[pl.ds(c * CH + t * L, L)] = raw & 15

        lane = lax.iota(jnp.int32, L)

        def row_body(kk, _):
            rowterm = (row0 + kk) * SH
            for c in range(nch):
                for t in range(CH // L):
                    gi_v[c, pl.ds(t * L, L)] = (
                        oih_v[c, pl.ds(t * L, L)] + rowterm
                    )
            ds = [
                pltpu.async_copy(
                    w_hbm.at[gi_v.at[c]], grow_v.at[pl.ds(c * CH, CH)], sem
                )
                for c in range(nch)
            ]
            for d in ds:
                d.wait()
            for t in range(R // L):
                vals = plsc.load_gather(
                    grow_v, [t * L + lane, oil_v[pl.ds(t * L, L)]]
                )
                rows_v[kk, pl.ds(t * L, L)] = vals
            return 0

        lax.fori_loop(0, rows_per, row_body, 0)
        pltpu.sync_copy(rows_v, wsub_hbm.at[pl.ds(row0, rows_per)])

        @pl.when(wid == 0)
        def _():
            ds = [
                pltpu.async_copy(
                    x_hbm.at[oih_v.at[c]], grow_v.at[pl.ds(c * CH, CH)], sem
                )
                for c in range(nch)
            ]
            for d in ds:
                d.wait()
            for t in range(R // L):
                vals = plsc.load_gather(
                    grow_v, [t * L + lane, oil_v[pl.ds(t * L, L)]]
                )
                xs_v[pl.ds(t * L, L)] = vals
            pltpu.sync_copy(xs_v, xsub_hbm)

    return k(w16, out_idx, x16)


def _tc_dense(weights, x, cw):
    R, S = weights.shape
    nsteps = S // cw

    def body(w_ref, x_ref, acc_ref):
        @pl.when(pl.program_id(0) == 0)
        def _():
            acc_ref[...] = jnp.zeros_like(acc_ref)

        acc_ref[...] += w_ref[...] * x_ref[...][None, :]

    return pl.pallas_call(
        body,
        grid=(nsteps,),
        in_specs=[
            pl.BlockSpec((R, cw), lambda i: (0, i)),
            pl.BlockSpec((cw,), lambda i: (i,)),
        ],
        out_specs=pl.BlockSpec((R, cw), lambda i: (0, 0)),
        out_shape=jax.ShapeDtypeStruct((R, cw), jnp.float32),
    )(weights, x)


def _tc_iterate(acc, b2, wsub, xs2, sg2, mi):
    R = wsub.shape[0]

    def body(acc_ref, b_ref, w_ref, xs_ref, sg_ref, mi_ref, out_ref):
        base = jnp.sum(acc_ref[...], axis=1, keepdims=True) + b_ref[...]
        xs = xs_ref[...]
        sg = sg_ref[...]
        dv = sg - xs
        w = w_ref[...]
        mi_v = mi_ref[0]

        def it(i, ef):
            delta = ef * dv
            act = base + jnp.dot(w, delta, preferred_element_type=jnp.float32)
            fired = (act > 0.0).astype(jnp.float32)
            ef2 = jnp.maximum(ef, fired)
            return jnp.where(i <= mi_v, ef2, ef)

        ef = lax.fori_loop(0, 20, it, jnp.zeros((R, 1), jnp.float32))
        out_ref[...] = jnp.where(ef > 0.0, sg, xs)

    return pl.pallas_call(
        body,
        in_specs=[
            pl.BlockSpec(memory_space=pltpu.VMEM),
            pl.BlockSpec(memory_space=pltpu.VMEM),
            pl.BlockSpec(memory_space=pltpu.VMEM),
            pl.BlockSpec(memory_space=pltpu.VMEM),
            pl.BlockSpec(memory_space=pltpu.VMEM),
            pl.BlockSpec(memory_space=pltpu.SMEM),
        ],
        out_specs=pl.BlockSpec(memory_space=pltpu.VMEM),
        out_shape=jax.ShapeDtypeStruct((R, 1), jnp.float32),
    )(acc, b2, wsub, xs2, sg2, mi)


def _sc_scatter(x, out_idx, vals):
    S = x.shape[0]
    R = out_idx.shape[0]
    per = S // NW

    mesh = plsc.VectorSubcoreMesh(core_axis_name="c", subcore_axis_name="s")

    @functools.partial(
        pl.kernel,
        out_type=jax.ShapeDtypeStruct((S,), jnp.float32),
        mesh=mesh,
        compiler_params=_SC_PARAMS,
        scratch_types=[
            pltpu.VMEM((per,), jnp.float32),
            pltpu.VMEM((R,), jnp.int32),
            pltpu.VMEM((R,), jnp.float32),
        ],
    )
    def k(x_hbm, oi_hbm, val_hbm, out_hbm, xb_v, oi_v, val_v):
        wid = lax.axis_index("s") * NC + lax.axis_index("c")
        base = wid * per
        pltpu.sync_copy(x_hbm.at[pl.ds(base, per)], xb_v)
        pltpu.sync_copy(oi_hbm, oi_v)
        pltpu.sync_copy(val_hbm, val_v)
        for t in range(R // L):
            idx = oi_v[pl.ds(t * L, L)]
            v = val_v[pl.ds(t * L, L)]
            loc = idx - base
            m = (loc >= 0) & (loc < per)
            locc = jnp.clip(loc, 0, per - 1)
            plsc.store_scatter(xb_v, [locc], v, mask=m)
        pltpu.sync_copy(xb_v, out_hbm.at[pl.ds(base, per)])

    return k(x, out_idx, vals)


def kernel(x, weights, biases, out_idx, out_sign, max_iters):
    R, S = weights.shape
    w16 = jnp.reshape(weights, (R * S // 16, 16))
    x16 = jnp.reshape(x, (S // 16, 16))
    wsub, xsub = _sc_gather(w16, out_idx, x16, R, S)
    acc = _tc_dense(weights, x, 1024)
    mi = jnp.reshape(jnp.asarray(max_iters, jnp.int32), (1,))
    vfin = _tc_iterate(
        acc,
        jnp.reshape(biases, (R, 1)),
        wsub,
        jnp.reshape(xsub, (R, 1)),
        jnp.reshape(out_sign, (R, 1)),
        mi,
    )
    return _sc_scatter(x, out_idx, jnp.reshape(vfin, (R,)))


# speedup vs baseline: 7.6802x; 2.0610x over previous
import functools

import jax
import jax.numpy as jnp
from jax import lax
from jax.experimental import pallas as pl
from jax.experimental.pallas import tpu as pltpu
from jax.experimental.pallas import tpu_sc as plsc

_SC_PARAMS = pltpu.CompilerParams(
    needs_layout_passes=False, use_tc_tiling_on_sc=False
)

NC = 2
NS = 16
NW = NC * NS
L = 16
CH = 128


def _sc_gather(wg, out_idx, x16, R, S):
    rows_per = R // NW
    nch = R // CH

    mesh = plsc.VectorSubcoreMesh(core_axis_name="c", subcore_axis_name="s")

    @functools.partial(
        pl.kernel,
        out_type=(
            jax.ShapeDtypeStruct((R, R), jnp.float32),
            jax.ShapeDtypeStruct((R,), jnp.float32),
        ),
        mesh=mesh,
        compiler_params=_SC_PARAMS,
        scratch_types=[
            pltpu.VMEM((nch, CH), jnp.int32),
            pltpu.VMEM((R,), jnp.int32),
            pltpu.VMEM((rows_per, nch, CH), jnp.int32),
            pltpu.VMEM((2, R, 16), jnp.float32),
            pltpu.VMEM((rows_per, R), jnp.float32),
            pltpu.VMEM((R,), jnp.float32),
            pltpu.SemaphoreType.DMA,
            pltpu.SemaphoreType.DMA,
        ],
    )
    def k(w_hbm, oi_hbm, x_hbm, wsub_hbm, xsub_hbm,
          cp_v, oil_v, gi_v, grow_v, rows_v, xs_v, sem_a, sem_b):
        wid = lax.axis_index("s") * NC + lax.axis_index("c")
        row0 = wid * rows_per
        for c in range(nch):
            pltpu.sync_copy(oi_hbm.at[pl.ds(c * CH, CH)], gi_v.at[0, c])
        for c in range(nch):
            for t in range(CH // L):
                raw = gi_v[0, c, pl.ds(t * L, L)]
                cp_v[c, pl.ds(t * L, L)] = ((raw >> 7) << 6) + ((raw >> 4) & 7)
                oil_v[pl.ds(c * CH + t * L, L)] = raw & 15

        def build(kk, _):
            r = row0 + kk
            rp = ((r >> 3) << 15) + ((kk & 7) << 3)
            for c in range(nch):
                for t in range(CH // L):
                    gi_v[kk, c, pl.ds(t * L, L)] = (
                        cp_v[c, pl.ds(t * L, L)] + rp
                    )
            return 0

        lax.fori_loop(0, rows_per, build, 0)

        lane = lax.iota(jnp.int32, L)

        def fire(kk, sl, sem):
            for c in range(nch):
                pltpu.async_copy(
                    w_hbm.at[gi_v.at[kk, c]],
                    grow_v.at[sl, pl.ds(c * CH, CH)],
                    sem,
                )

        def drain(sl, sem):
            pltpu.make_async_copy(
                w_hbm.at[pl.ds(0, R)], grow_v.at[sl], sem
            ).wait()

        def extract(kk, sl):
            for t in range(R // L):
                vals = plsc.load_gather(
                    grow_v.at[sl], [t * L + lane, oil_v[pl.ds(t * L, L)]]
                )
                rows_v[kk, pl.ds(t * L, L)] = vals

        fire(0, 0, sem_a)

        def pipe(m, _):
            kk0 = 2 * m
            fire(kk0 + 1, 1, sem_b)
            drain(0, sem_a)
            extract(kk0, 0)

            @pl.when(kk0 + 2 < rows_per)
            def _():
                fire(kk0 + 2, 0, sem_a)

            drain(1, sem_b)
            extract(kk0 + 1, 1)
            return 0

        lax.fori_loop(0, rows_per // 2, pipe, 0)
        pltpu.sync_copy(rows_v, wsub_hbm.at[pl.ds(row0, rows_per)])

        @pl.when(wid == 0)
        def _():
            for c in range(nch):
                for t in range(CH // L):
                    cp = cp_v[c, pl.ds(t * L, L)]
                    gi_v[0, c, pl.ds(t * L, L)] = ((cp >> 6) << 3) + (cp & 7)
            ds = [
                pltpu.async_copy(
                    x_hbm.at[gi_v.at[0, c]],
                    grow_v.at[0, pl.ds(c * CH, CH)],
                    sem_a,
                )
                for c in range(nch)
            ]
            for d in ds:
                d.wait()
            for t in range(R // L):
                vals = plsc.load_gather(
                    grow_v.at[0], [t * L + lane, oil_v[pl.ds(t * L, L)]]
                )
                xs_v[pl.ds(t * L, L)] = vals
            pltpu.sync_copy(xs_v, xsub_hbm)

    return k(wg, out_idx, x16)


def _tc_dense(weights, x, cw):
    R, S = weights.shape
    nsteps = S // cw

    def body(w_ref, x_ref, acc_ref):
        @pl.when(pl.program_id(0) == 0)
        def _():
            acc_ref[...] = jnp.zeros_like(acc_ref)

        acc_ref[...] += w_ref[...] * x_ref[...][None, :]

    return pl.pallas_call(
        body,
        grid=(nsteps,),
        in_specs=[
            pl.BlockSpec((R, cw), lambda i: (0, i)),
            pl.BlockSpec((cw,), lambda i: (i,)),
        ],
        out_specs=pl.BlockSpec((R, cw), lambda i: (0, 0)),
        out_shape=jax.ShapeDtypeStruct((R, cw), jnp.float32),
    )(weights, x)


def _tc_iterate(acc, b2, wsub, xs2, sg2, mi):
    R = wsub.shape[0]

    def body(acc_ref, b_ref, w_ref, xs_ref, sg_ref, mi_ref, out_ref):
        base = jnp.sum(acc_ref[...], axis=1, keepdims=True) + b_ref[...]
        xs = xs_ref[...]
        sg = sg_ref[...]
        dv = sg - xs
        w = w_ref[...]
        mi_v = mi_ref[0]

        def it(i, ef):
            delta = ef * dv
            act = base + jnp.dot(w, delta, preferred_element_type=jnp.float32)
            fired = (act > 0.0).astype(jnp.float32)
            ef2 = jnp.maximum(ef, fired)
            return jnp.where(i <= mi_v, ef2, ef)

        ef = lax.fori_loop(0, 20, it, jnp.zeros((R, 1), jnp.float32))
        out_ref[...] = jnp.where(ef > 0.0, sg, xs)

    return pl.pallas_call(
        body,
        in_specs=[
            pl.BlockSpec(memory_space=pltpu.VMEM),
            pl.BlockSpec(memory_space=pltpu.VMEM),
            pl.BlockSpec(memory_space=pltpu.VMEM),
            pl.BlockSpec(memory_space=pltpu.VMEM),
            pl.BlockSpec(memory_space=pltpu.VMEM),
            pl.BlockSpec(memory_space=pltpu.SMEM),
        ],
        out_specs=pl.BlockSpec(memory_space=pltpu.VMEM),
        out_shape=jax.ShapeDtypeStruct((R, 1), jnp.float32),
    )(acc, b2, wsub, xs2, sg2, mi)


def _sc_scatter(x, out_idx, vals):
    S = x.shape[0]
    R = out_idx.shape[0]
    per = S // NW

    mesh = plsc.VectorSubcoreMesh(core_axis_name="c", subcore_axis_name="s")

    @functools.partial(
        pl.kernel,
        out_type=jax.ShapeDtypeStruct((S,), jnp.float32),
        mesh=mesh,
        compiler_params=_SC_PARAMS,
        scratch_types=[
            pltpu.VMEM((per,), jnp.float32),
            pltpu.VMEM((R,), jnp.int32),
            pltpu.VMEM((R,), jnp.float32),
        ],
    )
    def k(x_hbm, oi_hbm, val_hbm, out_hbm, xb_v, oi_v, val_v):
        wid = lax.axis_index("s") * NC + lax.axis_index("c")
        base = wid * per
        pltpu.sync_copy(x_hbm.at[pl.ds(base, per)], xb_v)
        pltpu.sync_copy(oi_hbm, oi_v)
        pltpu.sync_copy(val_hbm, val_v)
        for t in range(R // L):
            idx = oi_v[pl.ds(t * L, L)]
            v = val_v[pl.ds(t * L, L)]
            loc = idx - base
            m = (loc >= 0) & (loc < per)
            locc = jnp.clip(loc, 0, per - 1)
            plsc.store_scatter(xb_v, [locc], v, mask=m)
        pltpu.sync_copy(xb_v, out_hbm.at[pl.ds(base, per)])

    return k(x, out_idx, vals)


def kernel(x, weights, biases, out_idx, out_sign, max_iters):
    R, S = weights.shape
    wg = jnp.reshape(
        jnp.transpose(
            jnp.reshape(weights, (R // 8, 8, S // 128, 128)), (0, 2, 1, 3)
        ),
        (R * S // 16, 16),
    )
    x16 = jnp.reshape(x, (S // 16, 16))
    wsub, xsub = _sc_gather(wg, out_idx, x16, R, S)
    acc = _tc_dense(weights, x, 1024)
    mi = jnp.reshape(jnp.asarray(max_iters, jnp.int32), (1,))
    vfin = _tc_iterate(
        acc,
        jnp.reshape(biases, (R, 1)),
        wsub,
        jnp.reshape(xsub, (R, 1)),
        jnp.reshape(out_sign, (R, 1)),
        mi,
    )
    return _sc_scatter(x, out_idx, jnp.reshape(vfin, (R,)))
